# baseline (speedup 1.0000x reference)
import functools

import jax
import jax.numpy as jnp
from jax import lax
from jax.experimental import pallas as pl
from jax.experimental.pallas import tpu as pltpu
from jax.experimental.pallas import tpu_sc as plsc

_NUM_CODES = 1024
_CODE_DIM = 256
_BETA = 0.25
_BM = 2304


def _csq_body(cb_ref, csq_ref):
    cb = cb_ref[...]
    csq_ref[...] = jnp.sum(cb * cb, axis=1)[None, :]


def _vq_body(x_ref, cb_ref, csq_ref, idx_ref, loss_ref, sse_ref):
    i = pl.program_id(0)
    n = pl.num_programs(0)
    cb = cb_ref[...]

    @pl.when(i == 0)
    def _init():
        sse_ref[...] = jnp.zeros_like(sse_ref)

    x = x_ref[...]
    zsq = jnp.sum(x * x, axis=1, keepdims=True)
    dot = jax.lax.dot_general(
        x, cb, (((1,), (1,)), ((), ())), preferred_element_type=jnp.float32
    )
    d = (zsq + csq_ref[...]) - 2.0 * dot
    m = jnp.min(d, axis=1, keepdims=True)
    iota = jax.lax.broadcasted_iota(jnp.int32, (_BM, _NUM_CODES), 1)
    idx = jnp.min(jnp.where(d == m, iota, _NUM_CODES), axis=1)
    idx_ref[...] = idx[None, None, :]

    sse_ref[...] = sse_ref[...] + jnp.sum(m)

    @pl.when(i == n - 1)
    def _fin():
        mse = sse_ref[0, 0] / (n * _BM * _CODE_DIM)
        loss_ref[...] = (mse + _BETA * mse)[None, None]


def _make_sc_gather(nrows):
    info = plsc.get_sparse_core_info()
    nw = info.num_cores * info.num_subcores
    b_per_w = nrows // nw
    mesh = plsc.VectorSubcoreMesh(core_axis_name="c", subcore_axis_name="s")

    @functools.partial(
        pl.kernel,
        mesh=mesh,
        out_type=jax.ShapeDtypeStruct((nrows, _CODE_DIM), jnp.float32),
        scratch_types=[
            pltpu.VMEM((b_per_w,), jnp.int32),
            pltpu.VMEM((b_per_w, _CODE_DIM), jnp.float32),
            pltpu.SemaphoreType.DMA,
        ],
    )
    def _gather(cb_hbm, idx_hbm, out_hbm, idx_v, rows_v, sem):
        wid = lax.axis_index("s") * info.num_cores + lax.axis_index("c")
        base = wid * b_per_w
        pltpu.sync_copy(idx_hbm.at[pl.ds(base, b_per_w)], idx_v)
        pltpu.async_copy(cb_hbm.at[idx_v], rows_v, sem).wait()
        pltpu.sync_copy(rows_v, out_hbm.at[pl.ds(base, b_per_w)])

    return _gather


def kernel(z, codebook):
    B, T, D = z.shape
    flat = z.reshape(-1, D)
    N = flat.shape[0]
    nblk = N // _BM

    csq = pl.pallas_call(
        _csq_body,
        out_shape=jax.ShapeDtypeStruct((1, _NUM_CODES), jnp.float32),
    )(codebook)

    idx3, loss = pl.pallas_call(
        _vq_body,
        grid=(nblk,),
        in_specs=[
            pl.BlockSpec((_BM, D), lambda i: (i, 0)),
            pl.BlockSpec((_NUM_CODES, D), lambda i: (0, 0)),
            pl.BlockSpec((1, _NUM_CODES), lambda i: (0, 0)),
        ],
        out_specs=[
            pl.BlockSpec((1, 1, _BM), lambda i: (i, 0, 0)),
            pl.BlockSpec((1, 1), lambda i: (0, 0)),
        ],
        out_shape=[
            jax.ShapeDtypeStruct((nblk, 1, _BM), jnp.int32),
            jax.ShapeDtypeStruct((1, 1), jnp.float32),
        ],
        scratch_shapes=[
            pltpu.VMEM((1, 1), jnp.float32),
        ],
    )(flat, codebook, csq)

    idx = idx3.reshape(N)
    qst = _make_sc_gather(N)(codebook, idx)

    return qst.reshape(B, T, D), loss[0, 0], idx3.reshape(B, T)


# speedup vs baseline: 1.4876x; 1.4876x over previous
import functools

import jax
import jax.numpy as jnp
from jax.experimental import pallas as pl
from jax.experimental.pallas import tpu as pltpu

_NUM_CODES = 1024
_CODE_DIM = 256
_BETA = 0.25
_BM = 2304


def _csq_body(cb_ref, csq_ref):
    cb = cb_ref[...]
    csq_ref[...] = jnp.sum(cb * cb, axis=1)[None, :]


def _vq_body(x_ref, cb_ref, csq_ref, cbh_ref, qst_ref, idx_ref, loss_ref, sse_ref):
    i = pl.program_id(0)
    n = pl.num_programs(0)
    cb = cb_ref[...]

    @pl.when(i == 0)
    def _init():
        sse_ref[...] = jnp.zeros_like(sse_ref)

    x = x_ref[...]
    zsq = jnp.sum(x * x, axis=1, keepdims=True)
    dot = jax.lax.dot_general(
        x, cb, (((1,), (1,)), ((), ())), preferred_element_type=jnp.float32
    )
    d = (zsq + csq_ref[...]) - 2.0 * dot
    m = jnp.min(d, axis=1, keepdims=True)
    iota = jax.lax.broadcasted_iota(jnp.int32, (_BM, _NUM_CODES), 1)
    idx = jnp.min(jnp.where(d == m, iota, _NUM_CODES), axis=1)
    idx_ref[...] = idx[None, None, :]

    onehot = (iota == idx[:, None]).astype(jnp.bfloat16)
    q = jax.lax.dot_general(
        onehot,
        cbh_ref[...],
        (((1,), (0,)), ((), ())),
        preferred_element_type=jnp.float32,
    )
    qst_ref[...] = q
    sse_ref[...] = sse_ref[...] + jnp.sum(m)

    @pl.when(i == n - 1)
    def _fin():
        mse = sse_ref[0, 0] / (n * _BM * _CODE_DIM)
        loss_ref[...] = (mse + _BETA * mse)[None, None]


def kernel(z, codebook):
    B, T, D = z.shape
    flat = z.reshape(-1, D)
    N = flat.shape[0]
    nblk = N // _BM

    csq = pl.pallas_call(
        _csq_body,
        out_shape=jax.ShapeDtypeStruct((1, _NUM_CODES), jnp.float32),
    )(codebook)

    qst, idx3, loss = pl.pallas_call(
        _vq_body,
        grid=(nblk,),
        in_specs=[
            pl.BlockSpec((_BM, D), lambda i: (i, 0)),
            pl.BlockSpec((_NUM_CODES, D), lambda i: (0, 0)),
            pl.BlockSpec((1, _NUM_CODES), lambda i: (0, 0)),
            pl.BlockSpec((_NUM_CODES, D), lambda i: (0, 0)),
        ],
        out_specs=[
            pl.BlockSpec((_BM, D), lambda i: (i, 0)),
            pl.BlockSpec((1, 1, _BM), lambda i: (i, 0, 0)),
            pl.BlockSpec((1, 1), lambda i: (0, 0)),
        ],
        out_shape=[
            jax.ShapeDtypeStruct((N, D), jnp.float32),
            jax.ShapeDtypeStruct((nblk, 1, _BM), jnp.int32),
            jax.ShapeDtypeStruct((1, 1), jnp.float32),
        ],
        scratch_shapes=[
            pltpu.VMEM((1, 1), jnp.float32),
        ],
    )(flat, codebook, csq, codebook.astype(jnp.bfloat16))

    return qst.reshape(B, T, D), loss[0, 0], idx3.reshape(B, T)


# speedup vs baseline: 1.5826x; 1.0638x over previous
import functools

import jax
import jax.numpy as jnp
from jax.experimental import pallas as pl
from jax.experimental.pallas import tpu as pltpu

_NUM_CODES = 1024
_CODE_DIM = 256
_BETA = 0.25
_BM = 2304


def _csq_body(cb_ref, csq_ref):
    cb = cb_ref[...]
    csq_ref[...] = jnp.sum(cb * cb, axis=1)[None, :]


def _vq_body(x_ref, cb_ref, csq_ref, qst_ref, idx_ref, loss_ref, sse_ref):
    i = pl.program_id(0)
    n = pl.num_programs(0)
    cb = cb_ref[...]

    @pl.when(i == 0)
    def _init():
        sse_ref[...] = jnp.zeros_like(sse_ref)

    x = x_ref[...]
    zsq = jnp.sum(x * x, axis=1, keepdims=True)
    dot = jax.lax.dot_general(
        x, cb, (((1,), (1,)), ((), ())), preferred_element_type=jnp.float32
    )
    d = (zsq + csq_ref[...]) - 2.0 * dot
    m = jnp.min(d, axis=1, keepdims=True)
    iota = jax.lax.broadcasted_iota(jnp.int32, (_BM, _NUM_CODES), 1)
    idx = jnp.min(jnp.where(d == m, iota, _NUM_CODES), axis=1)
    idx_ref[...] = idx[None, None, :]

    onehot = (iota == idx[:, None]).astype(jnp.float32)
    q = jax.lax.dot_general(
        onehot, cb, (((1,), (0,)), ((), ())), preferred_element_type=jnp.float32
    )
    qst_ref[...] = q
    sse_ref[...] = sse_ref[...] + jnp.sum(m)

    @pl.when(i == n - 1)
    def _fin():
        mse = sse_ref[0, 0] / (n * _BM * _CODE_DIM)
        loss_ref[...] = (mse + _BETA * mse)[None, None]


def kernel(z, codebook):
    B, T, D = z.shape
    flat = z.reshape(-1, D)
    N = flat.shape[0]
    nblk = N // _BM

    csq = pl.pallas_call(
        _csq_body,
        out_shape=jax.ShapeDtypeStruct((1, _NUM_CODES), jnp.float32),
    )(codebook)

    qst, idx3, loss = pl.pallas_call(
        _vq_body,
        grid=(nblk,),
        in_specs=[
            pl.BlockSpec((_BM, D), lambda i: (i, 0)),
            pl.BlockSpec((_NUM_CODES, D), lambda i: (0, 0)),
            pl.BlockSpec((1, _NUM_CODES), lambda i: (0, 0)),
        ],
        out_specs=[
            pl.BlockSpec((_BM, D), lambda i: (i, 0)),
            pl.BlockSpec((1, 1, _BM), lambda i: (i, 0, 0)),
            pl.BlockSpec((1, 1), lambda i: (0, 0)),
        ],
        out_shape=[
            jax.ShapeDtypeStruct((N, D), jnp.float32),
            jax.ShapeDtypeStruct((nblk, 1, _BM), jnp.int32),
            jax.ShapeDtypeStruct((1, 1), jnp.float32),
        ],
        scratch_shapes=[
            pltpu.VMEM((1, 1), jnp.float32),
        ],
    )(flat, codebook, csq)

    return qst.reshape(B, T, D), loss[0, 0], idx3.reshape(B, T)
